# Initial kernel scaffold; baseline (speedup 1.0000x reference)
import jax
import jax.numpy as jnp
from jax.experimental import pallas as pl


def kernel(input, W):
    raise NotImplementedError("write your pallas kernel here")


# speedup vs baseline: 1.2270x; 1.2270x over previous
import functools

import jax
import jax.numpy as jnp
from jax.experimental import pallas as pl
from jax.experimental.pallas import tpu as pltpu

NUM_EXPERTS = 64
TOKENS = 8192
MODEL_DIM = 2048
BLK = 512
NBLK = TOKENS // BLK


def _router_body(x_ref, w_ref, g_ref, idx_ref, loc_ref, loss_ref,
                 me_ref, ce_ref, carry_ref):
    i = pl.program_id(0)

    @pl.when(i == 0)
    def _init():
        me_ref[...] = jnp.zeros_like(me_ref)
        ce_ref[...] = jnp.zeros_like(ce_ref)
        carry_ref[...] = jnp.zeros_like(carry_ref)

    x = x_ref[...]
    w = w_ref[...]
    logits = jax.lax.dot_general(
        x, w, dimension_numbers=(((1,), (1,)), ((), ())),
        preferred_element_type=jnp.float32)

    m = jnp.max(logits, axis=1, keepdims=True)
    idx = jnp.argmax(logits, axis=1).astype(jnp.int32)
    e = jnp.exp(logits - m)
    s = jnp.sum(e, axis=1, keepdims=True)
    rs = 1.0 / s
    g_ref[...] = rs[:, 0]

    onehot = (jax.lax.broadcasted_iota(jnp.int32, (BLK, NUM_EXPERTS), 1)
              == idx[:, None]).astype(jnp.float32)

    me_ref[...] += jnp.sum(e * rs, axis=0, keepdims=True)
    ce_ref[...] += jnp.sum(onehot, axis=0, keepdims=True)

    r = jax.lax.broadcasted_iota(jnp.int32, (BLK, BLK), 0)
    c = jax.lax.broadcasted_iota(jnp.int32, (BLK, BLK), 1)
    tril = (c < r).astype(jnp.float32)
    excl = jax.lax.dot_general(
        tril, onehot, dimension_numbers=(((1,), (0,)), ((), ())),
        preferred_element_type=jnp.float32)
    within = jnp.sum(excl * onehot, axis=1)
    carried = jnp.sum(carry_ref[...] * onehot, axis=1)
    loc_ref[...] = (within + carried).astype(jnp.int32)
    carry_ref[...] += jnp.sum(onehot, axis=0, keepdims=True)

    idx_ref[...] = idx

    @pl.when(i == NBLK - 1)
    def _loss():
        me = me_ref[...]
        ce = ce_ref[...]
        loss = jnp.sum(me * ce) * (NUM_EXPERTS / (TOKENS * TOKENS))
        loss_ref[...] = jnp.full((1, 1), loss, jnp.float32)


@functools.partial(jax.jit, static_argnames=("interpret",))
def _router(x, w, interpret=False):
    return pl.pallas_call(
        _router_body,
        grid=(NBLK,),
        in_specs=[
            pl.BlockSpec((BLK, MODEL_DIM), lambda i: (i, 0)),
            pl.BlockSpec((NUM_EXPERTS, MODEL_DIM), lambda i: (0, 0)),
        ],
        out_specs=[
            pl.BlockSpec((BLK,), lambda i: (i,)),
            pl.BlockSpec((BLK,), lambda i: (i,)),
            pl.BlockSpec((BLK,), lambda i: (i,)),
            pl.BlockSpec((1, 1), lambda i: (0, 0)),
        ],
        out_shape=[
            jax.ShapeDtypeStruct((TOKENS,), jnp.float32),
            jax.ShapeDtypeStruct((TOKENS,), jnp.int32),
            jax.ShapeDtypeStruct((TOKENS,), jnp.int32),
            jax.ShapeDtypeStruct((1, 1), jnp.float32),
        ],
        scratch_shapes=[
            pltpu.VMEM((1, NUM_EXPERTS), jnp.float32),
            pltpu.VMEM((1, NUM_EXPERTS), jnp.float32),
            pltpu.VMEM((1, NUM_EXPERTS), jnp.float32),
        ],
        compiler_params=pltpu.CompilerParams(
            dimension_semantics=("arbitrary",)),
        interpret=interpret,
    )(x, w)


def kernel(input, W):
    g, idx, loc, loss = _router(input, W)
    return (loss[0, 0], g, idx, loc)
